# baseline (speedup 1.0000x reference)
import functools

import jax
import jax.numpy as jnp
from jax.experimental import pallas as pl

_BF = jnp.bfloat16
_I32 = jnp.int32
_F32 = jnp.float32


def _dot(a, b, dims):
    return jax.lax.dot_general(a, b, (dims, ((), ())),
                               preferred_element_type=_F32)


def _bit(x):
    return x.astype(_I32) & 1


def _syn_kernel(nz_ref, hx_ref, nx_ref, hz_ref, sxT_ref, szT_ref,
                pmx_ref, pmz_ref):
    bx = _bit(_dot(nz_ref[...], hx_ref[...], ((1,), (1,))))
    bz = _bit(_dot(nx_ref[...], hz_ref[...], ((1,), (1,))))
    sxT_ref[...] = bx.astype(_BF)
    szT_ref[...] = bz.astype(_BF)
    pmx_ref[...] = (1 - 2 * bx).astype(_BF)
    pmz_ref[...] = (1 - 2 * bz).astype(_BF)


def _bp_kernel(pmx_ref, hx_ref, pmz_ref, hz_ref, llr_ref, nx_ref, nz_ref,
               xd_ref, zd_ref, e0x_ref, e0z_ref):
    acc1 = _dot(pmx_ref[...], hx_ref[...], ((1,), (0,)))
    acc2 = _dot(pmz_ref[...], hz_ref[...], ((1,), (0,)))

    llr = llr_ref[...]
    llrz = llr + 0.1 * acc1
    llrx = llr + 0.1 * acc2
    llry = 0.5 * (llrx + llrz)
    xh = (llrx < 0.0).astype(_I32)
    zh = (llrz < 0.0).astype(_I32)
    xd_ref[...] = (nx_ref[...].astype(_I32) ^ xh).astype(_BF)
    zd_ref[...] = (nz_ref[...].astype(_I32) ^ zh).astype(_BF)

    def _lse2(a, b):
        m = jnp.maximum(a, b)
        return jnp.log(jnp.exp(a - m) + jnp.exp(b - m)) + m

    osd_z = jax.nn.softplus(-llrx) - _lse2(-llrz, -llry)
    osd_x = jax.nn.softplus(-llrz) - _lse2(-llrx, -llry)
    e0z_ref[...] = (osd_z < 0.0).astype(_BF)
    e0x_ref[...] = (osd_x < 0.0).astype(_BF)


def _res_kernel(xd_ref, hz_ref, zd_ref, hx_ref, err_ref, *, nj):
    j = pl.program_id(0)
    bits = _bit(_dot(xd_ref[...], hz_ref[...], ((1,), (1,)))) \
        + _bit(_dot(zd_ref[...], hx_ref[...], ((1,), (1,))))
    part = jnp.sum(bits, axis=1, keepdims=True)
    bcast = jnp.broadcast_to(part, err_ref.shape)

    @pl.when(j == 0)
    def _():
        err_ref[...] = bcast

    @pl.when(j != 0)
    def _():
        err_ref[...] += bcast


def _r_kernel(xz_ref, az_ref, xx_ref, ax_ref, rzT_ref, rxT_ref):
    rzT_ref[...] = _bit(_dot(xz_ref[...], az_ref[...], ((1,), (1,))))\
        .astype(_BF)
    rxT_ref[...] = _bit(_dot(xx_ref[...], ax_ref[...], ((1,), (1,))))\
        .astype(_BF)


def _fin_kernel(rzT_ref, hxb_ref, rxT_ref, hzb_ref, e0z_ref, e0x_ref,
                nx_ref, nz_ref, xd_ref, zd_ref, err_ref, lx_ref, lz_ref,
                lsx_ref, lsz_ref, *, nj):
    j = pl.program_id(0)
    acc1 = _dot(rzT_ref[...], hxb_ref[...], ((1,), (0,)))
    acc2 = _dot(rxT_ref[...], hzb_ref[...], ((1,), (0,)))

    errb = err_ref[...][:, :1] > 0
    ez = e0z_ref[...].astype(_I32) ^ _bit(acc1)
    ex = e0x_ref[...].astype(_I32) ^ _bit(acc2)
    d2z = jnp.where(errb, nz_ref[...].astype(_I32) ^ ez,
                    zd_ref[...].astype(_I32)).astype(_BF)
    d2x = jnp.where(errb, nx_ref[...].astype(_I32) ^ ex,
                    xd_ref[...].astype(_I32)).astype(_BF)

    partz = _dot(d2z, lx_ref[...], ((1,), (1,)))
    partx = _dot(d2x, lz_ref[...], ((1,), (1,)))

    @pl.when(j == 0)
    def _():
        lsz_ref[...] = partz
        lsx_ref[...] = partx

    @pl.when(j != 0)
    def _():
        lsz_ref[...] += partz
        lsx_ref[...] += partx

    @pl.when(j == nj - 1)
    def _():
        lsz_ref[...] = (lsz_ref[...].astype(_I32) & 1).astype(_F32)
        lsx_ref[...] = (lsx_ref[...].astype(_I32) & 1).astype(_F32)


def kernel(noise_x, noise_z, llr_ch, hx, hz, lx, lz, pivot_hx, pivot_hz,
           hx_basis, hz_basis):
    B, N = noise_x.shape
    M = hx.shape[0]
    RANK = hx_basis.shape[0]
    K = lx.shape[0]
    RP = ((RANK + 127) // 128) * 128
    CZ = N + M
    KP = 128
    BN = 512

    bf = _BF
    nx_b = noise_x.astype(bf)
    nz_b = noise_z.astype(bf)
    hx_b = hx.astype(bf)
    hz_b = hz.astype(bf)
    hxb_pad = jnp.pad(hx_basis, ((0, RP - RANK), (0, 0))).astype(bf)
    hzb_pad = jnp.pad(hz_basis, ((0, RP - RANK), (0, 0))).astype(bf)
    pvx = jnp.pad(pivot_hx, (0, RP - RANK), constant_values=-1)
    pvz = jnp.pad(pivot_hz, (0, RP - RANK), constant_values=-1)
    arange_m = jnp.arange(M, dtype=pvx.dtype)
    onehot_x = (pvx[:, None] == arange_m[None, :]).astype(bf)
    onehot_z = (pvz[:, None] == arange_m[None, :]).astype(bf)
    a_z = jnp.concatenate([hxb_pad, onehot_x], axis=1)
    a_x = jnp.concatenate([hzb_pad, onehot_z], axis=1)
    lx_pad = jnp.pad(lx, ((0, KP - K), (0, 0))).astype(bf)
    lz_pad = jnp.pad(lz, ((0, KP - K), (0, 0))).astype(bf)

    i32 = _I32
    f32 = _F32

    sxT, szT, pmx, pmz = pl.pallas_call(
        _syn_kernel,
        grid=(M // BN,),
        in_specs=[
            pl.BlockSpec((B, N), lambda j: (0, 0)),
            pl.BlockSpec((BN, N), lambda j: (j, 0)),
            pl.BlockSpec((B, N), lambda j: (0, 0)),
            pl.BlockSpec((BN, N), lambda j: (j, 0)),
        ],
        out_specs=[pl.BlockSpec((B, BN), lambda j: (0, j))] * 4,
        out_shape=[jax.ShapeDtypeStruct((B, M), bf)] * 4,
    )(nz_b, hx_b, nx_b, hz_b)

    xd, zd, e0x, e0z = pl.pallas_call(
        _bp_kernel,
        grid=(N // BN,),
        in_specs=[
            pl.BlockSpec((B, M), lambda j: (0, 0)),
            pl.BlockSpec((M, BN), lambda j: (0, j)),
            pl.BlockSpec((B, M), lambda j: (0, 0)),
            pl.BlockSpec((M, BN), lambda j: (0, j)),
            pl.BlockSpec((B, BN), lambda j: (0, j)),
            pl.BlockSpec((B, BN), lambda j: (0, j)),
            pl.BlockSpec((B, BN), lambda j: (0, j)),
        ],
        out_specs=[pl.BlockSpec((B, BN), lambda j: (0, j))] * 4,
        out_shape=[jax.ShapeDtypeStruct((B, N), bf)] * 4,
    )(pmx, hx_b, pmz, hz_b, llr_ch, nx_b, nz_b)

    nj = M // BN
    err_sum = pl.pallas_call(
        functools.partial(_res_kernel, nj=nj),
        grid=(nj,),
        in_specs=[
            pl.BlockSpec((B, N), lambda j: (0, 0)),
            pl.BlockSpec((BN, N), lambda j: (j, 0)),
            pl.BlockSpec((B, N), lambda j: (0, 0)),
            pl.BlockSpec((BN, N), lambda j: (j, 0)),
        ],
        out_specs=pl.BlockSpec((B, 128), lambda j: (0, 0)),
        out_shape=jax.ShapeDtypeStruct((B, 128), i32),
    )(xd, hz_b, zd, hx_b)

    x_z = jnp.concatenate([e0z, sxT], axis=1)
    x_x = jnp.concatenate([e0x, szT], axis=1)
    rzT, rxT = pl.pallas_call(
        _r_kernel,
        grid=(RP // BN,),
        in_specs=[
            pl.BlockSpec((B, CZ), lambda j: (0, 0)),
            pl.BlockSpec((BN, CZ), lambda j: (j, 0)),
            pl.BlockSpec((B, CZ), lambda j: (0, 0)),
            pl.BlockSpec((BN, CZ), lambda j: (j, 0)),
        ],
        out_specs=[pl.BlockSpec((B, BN), lambda j: (0, j))] * 2,
        out_shape=[jax.ShapeDtypeStruct((B, RP), bf)] * 2,
    )(x_z, a_z, x_x, a_x)

    nj = N // BN
    lsx_sum, lsz_sum = pl.pallas_call(
        functools.partial(_fin_kernel, nj=nj),
        grid=(nj,),
        in_specs=[
            pl.BlockSpec((B, RP), lambda j: (0, 0)),
            pl.BlockSpec((RP, BN), lambda j: (0, j)),
            pl.BlockSpec((B, RP), lambda j: (0, 0)),
            pl.BlockSpec((RP, BN), lambda j: (0, j)),
            pl.BlockSpec((B, BN), lambda j: (0, j)),
            pl.BlockSpec((B, BN), lambda j: (0, j)),
            pl.BlockSpec((B, BN), lambda j: (0, j)),
            pl.BlockSpec((B, BN), lambda j: (0, j)),
            pl.BlockSpec((B, BN), lambda j: (0, j)),
            pl.BlockSpec((B, BN), lambda j: (0, j)),
            pl.BlockSpec((B, 128), lambda j: (0, 0)),
            pl.BlockSpec((KP, BN), lambda j: (0, j)),
            pl.BlockSpec((KP, BN), lambda j: (0, j)),
        ],
        out_specs=[pl.BlockSpec((B, 128), lambda j: (0, 0))] * 2,
        out_shape=[jax.ShapeDtypeStruct((B, 128), f32)] * 2,
    )(rzT, hxb_pad, rxT, hzb_pad, e0z, e0x, nx_b, nz_b, xd, zd,
      err_sum, lx_pad, lz_pad)

    ls_hat = jnp.concatenate(
        [lsx_sum[:, :K], lsz_sum[:, :K]], axis=1).astype(jnp.int32)
    return (jnp.zeros_like(ls_hat), ls_hat)


# speedup vs baseline: 1.0845x; 1.0845x over previous
import functools

import jax
import jax.numpy as jnp
from jax.experimental import pallas as pl

_BF = jnp.bfloat16
_I32 = jnp.int32
_F32 = jnp.float32


def _dot(a, b, dims):
    return jax.lax.dot_general(a, b, (dims, ((), ())),
                               preferred_element_type=_F32)


def _bit(x):
    return x.astype(_I32) & 1


def _syn_kernel(nz_ref, hx_ref, nx_ref, hz_ref, sxT_ref, szT_ref,
                pmx_ref, pmz_ref):
    bx = _bit(_dot(nz_ref[...], hx_ref[...].astype(_BF), ((1,), (1,))))
    bz = _bit(_dot(nx_ref[...], hz_ref[...].astype(_BF), ((1,), (1,))))
    sxT_ref[...] = bx.astype(_BF)
    szT_ref[...] = bz.astype(_BF)
    pmx_ref[...] = (1 - 2 * bx).astype(_BF)
    pmz_ref[...] = (1 - 2 * bz).astype(_BF)


def _bp_kernel(pmx_ref, hx_ref, pmz_ref, hz_ref, llr_ref, nx_ref, nz_ref,
               xd_ref, zd_ref, e0x_ref, e0z_ref):
    acc1 = _dot(pmx_ref[...], hx_ref[...].astype(_BF), ((1,), (0,)))
    acc2 = _dot(pmz_ref[...], hz_ref[...].astype(_BF), ((1,), (0,)))

    llr = llr_ref[...]
    llrz = llr + 0.1 * acc1
    llrx = llr + 0.1 * acc2
    llry = 0.5 * (llrx + llrz)
    xh = (llrx < 0.0).astype(_I32)
    zh = (llrz < 0.0).astype(_I32)
    xd_ref[...] = (nx_ref[...].astype(_I32) ^ xh).astype(_BF)
    zd_ref[...] = (nz_ref[...].astype(_I32) ^ zh).astype(_BF)

    def _lse2(a, b):
        m = jnp.maximum(a, b)
        return jnp.log(jnp.exp(a - m) + jnp.exp(b - m)) + m

    osd_z = jax.nn.softplus(-llrx) - _lse2(-llrz, -llry)
    osd_x = jax.nn.softplus(-llrz) - _lse2(-llrx, -llry)
    e0z_ref[...] = (osd_z < 0.0).astype(_BF)
    e0x_ref[...] = (osd_x < 0.0).astype(_BF)


def _res_kernel(xd_ref, hz_ref, zd_ref, hx_ref, err_ref, *, nj):
    j = pl.program_id(0)
    bits = _bit(_dot(xd_ref[...], hz_ref[...].astype(_BF), ((1,), (1,)))) \
        + _bit(_dot(zd_ref[...], hx_ref[...].astype(_BF), ((1,), (1,))))
    part = jnp.sum(bits, axis=1, keepdims=True)
    bcast = jnp.broadcast_to(part, err_ref.shape)

    @pl.when(j == 0)
    def _():
        err_ref[...] = bcast

    @pl.when(j != 0)
    def _():
        err_ref[...] += bcast


def _r_kernel(xz_ref, az_ref, xx_ref, ax_ref, rzT_ref, rxT_ref):
    rzT_ref[...] = _bit(_dot(xz_ref[...], az_ref[...].astype(_BF), ((1,), (1,))))\
        .astype(_BF)
    rxT_ref[...] = _bit(_dot(xx_ref[...], ax_ref[...].astype(_BF), ((1,), (1,))))\
        .astype(_BF)


def _fin_kernel(rzT_ref, hxb_ref, rxT_ref, hzb_ref, e0z_ref, e0x_ref,
                nx_ref, nz_ref, xd_ref, zd_ref, err_ref, lx_ref, lz_ref,
                lsx_ref, lsz_ref, *, nj):
    j = pl.program_id(0)
    acc1 = _dot(rzT_ref[...], hxb_ref[...].astype(_BF), ((1,), (0,)))
    acc2 = _dot(rxT_ref[...], hzb_ref[...].astype(_BF), ((1,), (0,)))

    errb = err_ref[...][:, :1] > 0
    ez = e0z_ref[...].astype(_I32) ^ _bit(acc1)
    ex = e0x_ref[...].astype(_I32) ^ _bit(acc2)
    d2z = jnp.where(errb, nz_ref[...].astype(_I32) ^ ez,
                    zd_ref[...].astype(_I32)).astype(_BF)
    d2x = jnp.where(errb, nx_ref[...].astype(_I32) ^ ex,
                    xd_ref[...].astype(_I32)).astype(_BF)

    partz = _dot(d2z, lx_ref[...].astype(_BF), ((1,), (1,)))
    partx = _dot(d2x, lz_ref[...].astype(_BF), ((1,), (1,)))

    @pl.when(j == 0)
    def _():
        lsz_ref[...] = partz
        lsx_ref[...] = partx

    @pl.when(j != 0)
    def _():
        lsz_ref[...] += partz
        lsx_ref[...] += partx

    @pl.when(j == nj - 1)
    def _():
        lsz_ref[...] = (lsz_ref[...].astype(_I32) & 1).astype(_F32)
        lsx_ref[...] = (lsx_ref[...].astype(_I32) & 1).astype(_F32)


def kernel(noise_x, noise_z, llr_ch, hx, hz, lx, lz, pivot_hx, pivot_hz,
           hx_basis, hz_basis):
    B, N = noise_x.shape
    M = hx.shape[0]
    RANK = hx_basis.shape[0]
    K = lx.shape[0]
    RP = ((RANK + 127) // 128) * 128
    CZ = N + M
    KP = 128
    BN = 512

    bf = _BF
    i8 = jnp.int8
    nx_b = noise_x.astype(bf)
    nz_b = noise_z.astype(bf)
    hx_b = hx.astype(i8)
    hz_b = hz.astype(i8)
    hxb_pad = jnp.pad(hx_basis, ((0, RP - RANK), (0, 0))).astype(i8)
    hzb_pad = jnp.pad(hz_basis, ((0, RP - RANK), (0, 0))).astype(i8)
    pvx = jnp.pad(pivot_hx, (0, RP - RANK), constant_values=-1)
    pvz = jnp.pad(pivot_hz, (0, RP - RANK), constant_values=-1)
    arange_m = jnp.arange(M, dtype=pvx.dtype)
    onehot_x = (pvx[:, None] == arange_m[None, :]).astype(i8)
    onehot_z = (pvz[:, None] == arange_m[None, :]).astype(i8)
    a_z = jnp.concatenate([hxb_pad, onehot_x], axis=1)
    a_x = jnp.concatenate([hzb_pad, onehot_z], axis=1)
    lx_pad = jnp.pad(lx, ((0, KP - K), (0, 0))).astype(i8)
    lz_pad = jnp.pad(lz, ((0, KP - K), (0, 0))).astype(i8)

    i32 = _I32
    f32 = _F32
    del i8

    sxT, szT, pmx, pmz = pl.pallas_call(
        _syn_kernel,
        grid=(M // BN,),
        in_specs=[
            pl.BlockSpec((B, N), lambda j: (0, 0)),
            pl.BlockSpec((BN, N), lambda j: (j, 0)),
            pl.BlockSpec((B, N), lambda j: (0, 0)),
            pl.BlockSpec((BN, N), lambda j: (j, 0)),
        ],
        out_specs=[pl.BlockSpec((B, BN), lambda j: (0, j))] * 4,
        out_shape=[jax.ShapeDtypeStruct((B, M), bf)] * 4,
    )(nz_b, hx_b, nx_b, hz_b)

    xd, zd, e0x, e0z = pl.pallas_call(
        _bp_kernel,
        grid=(N // BN,),
        in_specs=[
            pl.BlockSpec((B, M), lambda j: (0, 0)),
            pl.BlockSpec((M, BN), lambda j: (0, j)),
            pl.BlockSpec((B, M), lambda j: (0, 0)),
            pl.BlockSpec((M, BN), lambda j: (0, j)),
            pl.BlockSpec((B, BN), lambda j: (0, j)),
            pl.BlockSpec((B, BN), lambda j: (0, j)),
            pl.BlockSpec((B, BN), lambda j: (0, j)),
        ],
        out_specs=[pl.BlockSpec((B, BN), lambda j: (0, j))] * 4,
        out_shape=[jax.ShapeDtypeStruct((B, N), bf)] * 4,
    )(pmx, hx_b, pmz, hz_b, llr_ch, nx_b, nz_b)

    nj = M // BN
    err_sum = pl.pallas_call(
        functools.partial(_res_kernel, nj=nj),
        grid=(nj,),
        in_specs=[
            pl.BlockSpec((B, N), lambda j: (0, 0)),
            pl.BlockSpec((BN, N), lambda j: (j, 0)),
            pl.BlockSpec((B, N), lambda j: (0, 0)),
            pl.BlockSpec((BN, N), lambda j: (j, 0)),
        ],
        out_specs=pl.BlockSpec((B, 128), lambda j: (0, 0)),
        out_shape=jax.ShapeDtypeStruct((B, 128), i32),
    )(xd, hz_b, zd, hx_b)

    x_z = jnp.concatenate([e0z, sxT], axis=1)
    x_x = jnp.concatenate([e0x, szT], axis=1)
    rzT, rxT = pl.pallas_call(
        _r_kernel,
        grid=(RP // BN,),
        in_specs=[
            pl.BlockSpec((B, CZ), lambda j: (0, 0)),
            pl.BlockSpec((BN, CZ), lambda j: (j, 0)),
            pl.BlockSpec((B, CZ), lambda j: (0, 0)),
            pl.BlockSpec((BN, CZ), lambda j: (j, 0)),
        ],
        out_specs=[pl.BlockSpec((B, BN), lambda j: (0, j))] * 2,
        out_shape=[jax.ShapeDtypeStruct((B, RP), bf)] * 2,
    )(x_z, a_z, x_x, a_x)

    nj = N // BN
    lsx_sum, lsz_sum = pl.pallas_call(
        functools.partial(_fin_kernel, nj=nj),
        grid=(nj,),
        in_specs=[
            pl.BlockSpec((B, RP), lambda j: (0, 0)),
            pl.BlockSpec((RP, BN), lambda j: (0, j)),
            pl.BlockSpec((B, RP), lambda j: (0, 0)),
            pl.BlockSpec((RP, BN), lambda j: (0, j)),
            pl.BlockSpec((B, BN), lambda j: (0, j)),
            pl.BlockSpec((B, BN), lambda j: (0, j)),
            pl.BlockSpec((B, BN), lambda j: (0, j)),
            pl.BlockSpec((B, BN), lambda j: (0, j)),
            pl.BlockSpec((B, BN), lambda j: (0, j)),
            pl.BlockSpec((B, BN), lambda j: (0, j)),
            pl.BlockSpec((B, 128), lambda j: (0, 0)),
            pl.BlockSpec((KP, BN), lambda j: (0, j)),
            pl.BlockSpec((KP, BN), lambda j: (0, j)),
        ],
        out_specs=[pl.BlockSpec((B, 128), lambda j: (0, 0))] * 2,
        out_shape=[jax.ShapeDtypeStruct((B, 128), f32)] * 2,
    )(rzT, hxb_pad, rxT, hzb_pad, e0z, e0x, nx_b, nz_b, xd, zd,
      err_sum, lx_pad, lz_pad)

    ls_hat = jnp.concatenate(
        [lsx_sum[:, :K], lsz_sum[:, :K]], axis=1).astype(jnp.int32)
    return (jnp.zeros_like(ls_hat), ls_hat)


# speedup vs baseline: 1.1165x; 1.0295x over previous
import functools

import jax
import jax.numpy as jnp
from jax.experimental import pallas as pl

_I8 = jnp.int8
_I32 = jnp.int32
_F32 = jnp.float32


def _dot(a, b, dims):
    return jax.lax.dot_general(a, b, (dims, ((), ())),
                               preferred_element_type=_I32)


def _syn_kernel(nz_ref, hx_ref, nx_ref, hz_ref, sxT_ref, szT_ref,
                pmx_ref, pmz_ref):
    bx = _dot(nz_ref[...], hx_ref[...], ((1,), (1,))) & 1
    bz = _dot(nx_ref[...], hz_ref[...], ((1,), (1,))) & 1
    sxT_ref[...] = bx.astype(_I8)
    szT_ref[...] = bz.astype(_I8)
    pmx_ref[...] = (1 - 2 * bx).astype(_I8)
    pmz_ref[...] = (1 - 2 * bz).astype(_I8)


def _bp_kernel(pmx_ref, hx_ref, pmz_ref, hz_ref, llr_ref, nx_ref, nz_ref,
               xd_ref, zd_ref, e0x_ref, e0z_ref):
    acc1 = _dot(pmx_ref[...], hx_ref[...], ((1,), (0,)))
    acc2 = _dot(pmz_ref[...], hz_ref[...], ((1,), (0,)))

    llr = llr_ref[...]
    llrz = llr + 0.1 * acc1.astype(_F32)
    llrx = llr + 0.1 * acc2.astype(_F32)
    llry = 0.5 * (llrx + llrz)
    xh = (llrx < 0.0).astype(_I32)
    zh = (llrz < 0.0).astype(_I32)
    xd_ref[...] = (nx_ref[...].astype(_I32) ^ xh).astype(_I8)
    zd_ref[...] = (nz_ref[...].astype(_I32) ^ zh).astype(_I8)

    def _lse2(a, b):
        m = jnp.maximum(a, b)
        return jnp.log(jnp.exp(a - m) + jnp.exp(b - m)) + m

    osd_z = jax.nn.softplus(-llrx) - _lse2(-llrz, -llry)
    osd_x = jax.nn.softplus(-llrz) - _lse2(-llrx, -llry)
    e0z_ref[...] = (osd_z < 0.0).astype(_I32).astype(_I8)
    e0x_ref[...] = (osd_x < 0.0).astype(_I32).astype(_I8)


def _res_kernel(xd_ref, hz_ref, zd_ref, hx_ref, err_ref, *, nj):
    j = pl.program_id(0)
    bits = (_dot(xd_ref[...], hz_ref[...], ((1,), (1,))) & 1) \
        + (_dot(zd_ref[...], hx_ref[...], ((1,), (1,))) & 1)
    part = jnp.sum(bits, axis=1, keepdims=True)
    bcast = jnp.broadcast_to(part, err_ref.shape)

    @pl.when(j == 0)
    def _():
        err_ref[...] = bcast

    @pl.when(j != 0)
    def _():
        err_ref[...] += bcast


def _r_kernel(xz_ref, az_ref, xx_ref, ax_ref, rzT_ref, rxT_ref):
    rzT_ref[...] = (_dot(xz_ref[...], az_ref[...], ((1,), (1,))) & 1)\
        .astype(_I8)
    rxT_ref[...] = (_dot(xx_ref[...], ax_ref[...], ((1,), (1,))) & 1)\
        .astype(_I8)


def _fin_kernel(rzT_ref, hxb_ref, rxT_ref, hzb_ref, e0z_ref, e0x_ref,
                nx_ref, nz_ref, xd_ref, zd_ref, err_ref, lx_ref, lz_ref,
                lsx_ref, lsz_ref, *, nj):
    j = pl.program_id(0)
    acc1 = _dot(rzT_ref[...], hxb_ref[...], ((1,), (0,)))
    acc2 = _dot(rxT_ref[...], hzb_ref[...], ((1,), (0,)))

    errb = err_ref[...][:, :1] > 0
    ez = e0z_ref[...].astype(_I32) ^ (acc1 & 1)
    ex = e0x_ref[...].astype(_I32) ^ (acc2 & 1)
    d2z = jnp.where(errb, nz_ref[...].astype(_I32) ^ ez,
                    zd_ref[...].astype(_I32)).astype(_I8)
    d2x = jnp.where(errb, nx_ref[...].astype(_I32) ^ ex,
                    xd_ref[...].astype(_I32)).astype(_I8)

    partz = _dot(d2z, lx_ref[...], ((1,), (1,)))
    partx = _dot(d2x, lz_ref[...], ((1,), (1,)))

    @pl.when(j == 0)
    def _():
        lsz_ref[...] = partz
        lsx_ref[...] = partx

    @pl.when(j != 0)
    def _():
        lsz_ref[...] += partz
        lsx_ref[...] += partx

    @pl.when(j == nj - 1)
    def _():
        lsz_ref[...] = lsz_ref[...] & 1
        lsx_ref[...] = lsx_ref[...] & 1


def kernel(noise_x, noise_z, llr_ch, hx, hz, lx, lz, pivot_hx, pivot_hz,
           hx_basis, hz_basis):
    B, N = noise_x.shape
    M = hx.shape[0]
    RANK = hx_basis.shape[0]
    K = lx.shape[0]
    RP = ((RANK + 127) // 128) * 128
    CZ = N + M
    KP = 128
    BN = 1024

    i8 = _I8
    nx_b = noise_x.astype(i8)
    nz_b = noise_z.astype(i8)
    hx_b = hx.astype(i8)
    hz_b = hz.astype(i8)
    hxb_pad = jnp.pad(hx_basis, ((0, RP - RANK), (0, 0))).astype(i8)
    hzb_pad = jnp.pad(hz_basis, ((0, RP - RANK), (0, 0))).astype(i8)
    pvx = jnp.pad(pivot_hx, (0, RP - RANK), constant_values=-1)
    pvz = jnp.pad(pivot_hz, (0, RP - RANK), constant_values=-1)
    arange_m = jnp.arange(M, dtype=pvx.dtype)
    onehot_x = (pvx[:, None] == arange_m[None, :]).astype(i8)
    onehot_z = (pvz[:, None] == arange_m[None, :]).astype(i8)
    a_z = jnp.concatenate([hxb_pad, onehot_x], axis=1)
    a_x = jnp.concatenate([hzb_pad, onehot_z], axis=1)
    lx_pad = jnp.pad(lx, ((0, KP - K), (0, 0))).astype(i8)
    lz_pad = jnp.pad(lz, ((0, KP - K), (0, 0))).astype(i8)

    i32 = _I32

    sxT, szT, pmx, pmz = pl.pallas_call(
        _syn_kernel,
        grid=(M // BN,),
        in_specs=[
            pl.BlockSpec((B, N), lambda j: (0, 0)),
            pl.BlockSpec((BN, N), lambda j: (j, 0)),
            pl.BlockSpec((B, N), lambda j: (0, 0)),
            pl.BlockSpec((BN, N), lambda j: (j, 0)),
        ],
        out_specs=[pl.BlockSpec((B, BN), lambda j: (0, j))] * 4,
        out_shape=[jax.ShapeDtypeStruct((B, M), i8)] * 4,
    )(nz_b, hx_b, nx_b, hz_b)

    xd, zd, e0x, e0z = pl.pallas_call(
        _bp_kernel,
        grid=(N // BN,),
        in_specs=[
            pl.BlockSpec((B, M), lambda j: (0, 0)),
            pl.BlockSpec((M, BN), lambda j: (0, j)),
            pl.BlockSpec((B, M), lambda j: (0, 0)),
            pl.BlockSpec((M, BN), lambda j: (0, j)),
            pl.BlockSpec((B, BN), lambda j: (0, j)),
            pl.BlockSpec((B, BN), lambda j: (0, j)),
            pl.BlockSpec((B, BN), lambda j: (0, j)),
        ],
        out_specs=[pl.BlockSpec((B, BN), lambda j: (0, j))] * 4,
        out_shape=[jax.ShapeDtypeStruct((B, N), i8)] * 4,
    )(pmx, hx_b, pmz, hz_b, llr_ch, nx_b, nz_b)

    nj = M // BN
    err_sum = pl.pallas_call(
        functools.partial(_res_kernel, nj=nj),
        grid=(nj,),
        in_specs=[
            pl.BlockSpec((B, N), lambda j: (0, 0)),
            pl.BlockSpec((BN, N), lambda j: (j, 0)),
            pl.BlockSpec((B, N), lambda j: (0, 0)),
            pl.BlockSpec((BN, N), lambda j: (j, 0)),
        ],
        out_specs=pl.BlockSpec((B, 128), lambda j: (0, 0)),
        out_shape=jax.ShapeDtypeStruct((B, 128), i32),
    )(xd, hz_b, zd, hx_b)

    x_z = jnp.concatenate([e0z, sxT], axis=1)
    x_x = jnp.concatenate([e0x, szT], axis=1)
    rzT, rxT = pl.pallas_call(
        _r_kernel,
        grid=(RP // BN,),
        in_specs=[
            pl.BlockSpec((B, CZ), lambda j: (0, 0)),
            pl.BlockSpec((BN, CZ), lambda j: (j, 0)),
            pl.BlockSpec((B, CZ), lambda j: (0, 0)),
            pl.BlockSpec((BN, CZ), lambda j: (j, 0)),
        ],
        out_specs=[pl.BlockSpec((B, BN), lambda j: (0, j))] * 2,
        out_shape=[jax.ShapeDtypeStruct((B, RP), i8)] * 2,
    )(x_z, a_z, x_x, a_x)

    nj = N // BN
    lsx_sum, lsz_sum = pl.pallas_call(
        functools.partial(_fin_kernel, nj=nj),
        grid=(nj,),
        in_specs=[
            pl.BlockSpec((B, RP), lambda j: (0, 0)),
            pl.BlockSpec((RP, BN), lambda j: (0, j)),
            pl.BlockSpec((B, RP), lambda j: (0, 0)),
            pl.BlockSpec((RP, BN), lambda j: (0, j)),
            pl.BlockSpec((B, BN), lambda j: (0, j)),
            pl.BlockSpec((B, BN), lambda j: (0, j)),
            pl.BlockSpec((B, BN), lambda j: (0, j)),
            pl.BlockSpec((B, BN), lambda j: (0, j)),
            pl.BlockSpec((B, BN), lambda j: (0, j)),
            pl.BlockSpec((B, BN), lambda j: (0, j)),
            pl.BlockSpec((B, 128), lambda j: (0, 0)),
            pl.BlockSpec((KP, BN), lambda j: (0, j)),
            pl.BlockSpec((KP, BN), lambda j: (0, j)),
        ],
        out_specs=[pl.BlockSpec((B, 128), lambda j: (0, 0))] * 2,
        out_shape=[jax.ShapeDtypeStruct((B, 128), i32)] * 2,
    )(rzT, hxb_pad, rxT, hzb_pad, e0z, e0x, nx_b, nz_b, xd, zd,
      err_sum, lx_pad, lz_pad)

    ls_hat = jnp.concatenate([lsx_sum[:, :K], lsz_sum[:, :K]], axis=1)
    return (jnp.zeros_like(ls_hat), ls_hat)
